# baseline (speedup 1.0000x reference)
import functools

import jax
import jax.numpy as jnp
from jax import lax
from jax.experimental import pallas as pl
from jax.experimental.pallas import tpu as pltpu
from jax.experimental.pallas import tpu_sc as plsc

N = 10000
E = 160000
D_IN = 256
D_H = 256
DQ = 64
N_CLASSES = 16

NC = 2
NS = 16
B = 128
RPT = 80
NCHUNK = NS * RPT
EPAD = NCHUNK * B - E
RLAST = RPT - EPAD // B
NDUMP = 8
NSP = N + NDUMP
NBUF = 2
NSTEP = RPT // NBUF
RSTRIPE = 624
NREM = N - RSTRIPE * NS

_mesh = plsc.VectorSubcoreMesh(
    core_axis_name="c", subcore_axis_name="s", num_cores=NC, num_subcores=NS
)


def _zero_stripe(zbuf, sh_ref, s):
    base = s * RSTRIPE
    for k in range(4):
        pltpu.sync_copy(zbuf, sh_ref.at[pl.ds(base + k * B, B)])
    pltpu.sync_copy(zbuf.at[pl.ds(0, RSTRIPE - 4 * B)],
                    sh_ref.at[pl.ds(base + 4 * B, RSTRIPE - 4 * B)])

    @pl.when(s == NS - 1)
    def _():
        pltpu.sync_copy(zbuf.at[pl.ds(0, NREM)], sh_ref.at[pl.ds(NS * RSTRIPE, NREM)])


def _copy_stripe(sh_ref, out_ref, s):
    base = s * RSTRIPE
    pltpu.sync_copy(sh_ref.at[pl.ds(base, RSTRIPE)], out_ref.at[pl.ds(base, RSTRIPE)])

    @pl.when(s == NS - 1)
    def _():
        pltpu.sync_copy(sh_ref.at[pl.ds(NS * RSTRIPE, NREM)],
                        out_ref.at[pl.ds(NS * RSTRIPE, NREM)])


def _fill(ref, rows, val):
    v = jnp.full((16,), val, dtype=ref.dtype)

    def body(i, _):
        ref[i, :] = v
        return 0

    lax.fori_loop(0, rows, body, 0)


@functools.partial(
    pl.kernel,
    out_type=(
        jax.ShapeDtypeStruct((N, 16), jnp.float32),
        jax.ShapeDtypeStruct((N, 16), jnp.float32),
    ),
    mesh=_mesh,
    scratch_types=[
        pltpu.VMEM((RPT, B), jnp.int32),
        pltpu.VMEM((B, 16), jnp.float32),
        pltpu.VMEM((B, 16), jnp.float32),
        pltpu.VMEM_SHARED((NSP, 16), jnp.float32),
    ] + [pltpu.SemaphoreType.DMA] * 8,
)
def _deg_kernel(srch2d, dsth2d, hsrc, hdst, idx_all, ones_v, zeros_v, hist_sh,
                *sems):
    c = lax.axis_index("c")
    s = lax.axis_index("s")

    _fill(ones_v, B, 1.0)
    _fill(zeros_v, B, 0.0)
    _zero_stripe(zeros_v, hist_sh, s)

    def scan_edges(edges_ref, out_ref):
        pltpu.sync_copy(edges_ref.at[pl.ds(s * RPT, RPT)], idx_all)
        plsc.subcore_barrier()

        pend = [None] * 8
        for r in range(RPT):
            p = r % 8
            if pend[p] is not None:
                pend[p].wait()
            pend[p] = pltpu.async_copy(
                ones_v, hist_sh.at[idx_all.at[r]], sems[p], add=True)
        for p in range(8):
            pend[p].wait()

        plsc.subcore_barrier()
        _copy_stripe(hist_sh, out_ref, s)

    @pl.when(c == 0)
    def _():
        scan_edges(srch2d, hsrc)

    @pl.when(c == 1)
    def _():
        scan_edges(dsth2d, hdst)


@functools.partial(
    pl.kernel,
    out_type=jax.ShapeDtypeStruct((4 * N, DQ), jnp.float32),
    mesh=_mesh,
    compiler_params=pltpu.CompilerParams(use_tc_tiling_on_sc=False),
    scratch_types=[
        pltpu.VMEM((RPT, B), jnp.int32),
        pltpu.VMEM((RPT, B), jnp.int32),
    ] + [pltpu.VMEM((B, DQ), jnp.float32)] * 8
      + [pltpu.VMEM_SHARED((NSP, DQ), jnp.float32)]
      + [pltpu.SemaphoreType.DMA] * 14,
)
def _agg_kernel(srcg2d, dsth2d, y4, agg4, sidx_all, didx_all, *scr):
    rows = list(scr[0:7])
    zrow_v = scr[7]
    agg_sh = scr[8]
    gsem = list(scr[9:16])
    ssem = list(scr[16:23])
    c = lax.axis_index("c")
    s = lax.axis_index("s")

    def zc(i, _):
        for k in range(DQ // 16):
            zrow_v[i, pl.ds(k * 16, 16)] = jnp.zeros((16,), jnp.float32)
        return 0

    lax.fori_loop(0, B, zc, 0)

    pltpu.sync_copy(dsth2d.at[pl.ds(s * RPT, RPT)], didx_all)

    def gather(r, q):
        return pltpu.async_copy(y4.at[sidx_all.at[r]], rows[q], gsem[q])

    def scat(r, q):
        return pltpu.async_copy(
            rows[q], agg_sh.at[didx_all.at[r]], ssem[q], add=True)

    for p in range(2):
        _zero_stripe(zrow_v, agg_sh, s)

        pltpu.sync_copy(srcg2d.at[pl.ds(s * RPT, RPT)], sidx_all)
        off = (2 * c + p) * N

        def addoff(i, _):
            for k in range(B // 16):
                sl = pl.ds(k * 16, 16)
                sidx_all[i, sl] = sidx_all[i, sl] + off
            return 0

        lax.fori_loop(0, RPT, addoff, 0)

        plsc.subcore_barrier()

        gd = [None] * RPT
        sd = [None] * RPT
        for r in range(RPT):
            if r >= 7:
                sd[r - 7].wait()
            gd[r] = gather(r, r % 7)
            if r >= 3:
                gd[r - 3].wait()
                sd[r - 3] = scat(r - 3, (r - 3) % 7)
        for r in range(RPT - 3, RPT):
            gd[r].wait()
            sd[r] = scat(r, r % 7)
        for r in range(RPT - 7, RPT):
            sd[r].wait()

        plsc.subcore_barrier()
        base = s * RSTRIPE
        pltpu.sync_copy(agg_sh.at[pl.ds(base, RSTRIPE)],
                        agg4.at[pl.ds(off + base, RSTRIPE)])

        @pl.when(s == NS - 1)
        def _(off=off):
            pltpu.sync_copy(agg_sh.at[pl.ds(NS * RSTRIPE, NREM)],
                            agg4.at[pl.ds(off + NS * RSTRIPE, NREM)])


_RB = 1000


def _mm_body(xf_ref, w1_ref, hs_ref, y_ref):
    x = jax.lax.dot_general(
        xf_ref[0], w1_ref[0], (((1,), (0,)), ((), ())),
        preferred_element_type=jnp.float32,
        precision=jax.lax.Precision.DEFAULT,
    )
    deg = jnp.maximum(hs_ref[:, 0], 1.0)
    ns = jax.lax.rsqrt(deg)[:, None]
    y_ref[0] = x * ns


def _mm_scale(in_feat, W1, hsrc):
    w1q = W1.reshape(D_IN, 4, DQ).transpose(1, 0, 2)
    y4 = pl.pallas_call(
        _mm_body,
        grid=(4, N // _RB),
        in_specs=[
            pl.BlockSpec((1, _RB, D_IN), lambda j, i: (0, i, 0)),
            pl.BlockSpec((1, D_IN, DQ), lambda j, i: (j, 0, 0)),
            pl.BlockSpec((_RB, 16), lambda j, i: (i, 0)),
        ],
        out_specs=pl.BlockSpec((1, _RB, DQ), lambda j, i: (j, i, 0)),
        out_shape=jax.ShapeDtypeStruct((4, N, DQ), jnp.float32),
    )(in_feat.reshape(1, N, D_IN), w1q, hsrc)
    return y4.reshape(4 * N, DQ)


def _epi_body(a0_ref, a1_ref, a2_ref, a3_ref, hd_ref, b1_ref, w2_ref, b2_ref,
              out_ref, acc_ref):
    i = pl.program_id(0)
    deg = jnp.maximum(hd_ref[:, 0], 1.0)
    nd = jax.lax.rsqrt(deg)[:, None]
    aq = [a0_ref, a1_ref, a2_ref, a3_ref]
    hq = [aq[q][...] * nd + b1_ref[0:1, DQ * q:DQ * (q + 1)] for q in range(4)]
    ss = sum(jnp.sum(h * h, axis=1, keepdims=True) for h in hq)
    inv = 1.0 / jnp.maximum(jnp.sqrt(ss), 1e-12)
    sq = [jax.nn.sigmoid(h * inv) for h in hq]

    @pl.when(i == 0)
    def _():
        acc_ref[...] = jnp.zeros_like(acc_ref)

    for q in range(4):
        acc_ref[0:1, DQ * q:DQ * (q + 1)] += jnp.sum(sq[q], axis=0, keepdims=True)

    @pl.when(i == (N // _RB) - 1)
    def _():
        hg = acc_ref[...] * (1.0 / N)
        out_ref[...] = (
            jax.lax.dot_general(
                hg, w2_ref[...], (((1,), (0,)), ((), ())),
                preferred_element_type=jnp.float32,
                precision=jax.lax.Precision.DEFAULT,
            )
            + b2_ref[...]
        )


def _epilogue(agg4, hdst, b1, W2, b2):
    qspec = pl.BlockSpec((_RB, DQ), lambda i: (i, 0))
    qspecs = [
        pl.BlockSpec((_RB, DQ), lambda i, q=q: (q * (N // _RB) + i, 0))
        for q in range(4)
    ]
    return pl.pallas_call(
        _epi_body,
        grid=(N // _RB,),
        in_specs=qspecs + [
            pl.BlockSpec((_RB, 16), lambda i: (i, 0)),
            pl.BlockSpec((1, D_H), lambda i: (0, 0)),
            pl.BlockSpec((D_H, N_CLASSES), lambda i: (0, 0)),
            pl.BlockSpec((1, N_CLASSES), lambda i: (0, 0)),
        ],
        out_specs=pl.BlockSpec((1, N_CLASSES), lambda i: (0, 0)),
        out_shape=jax.ShapeDtypeStruct((1, N_CLASSES), jnp.float32),
        scratch_shapes=[pltpu.VMEM((1, D_H), jnp.float32)],
    )(agg4, agg4, agg4, agg4, hdst, b1.reshape(1, D_H), W2,
      b2.reshape(1, N_CLASSES))


def kernel(in_feat, edge_index, W1, b1, W2, b2):
    ar = jnp.arange(EPAD, dtype=jnp.int32)
    dump = N + (ar % NDUMP)
    srch2d = jnp.concatenate([edge_index[0], dump]).reshape(NCHUNK, B)
    srcg2d = jnp.concatenate([edge_index[0], ar % B]).reshape(NCHUNK, B)
    dsth2d = jnp.concatenate([edge_index[1], dump]).reshape(NCHUNK, B)
    hsrc, hdst = _deg_kernel(srch2d, dsth2d)
    y4 = _mm_scale(in_feat, W1, hsrc)
    agg4 = _agg_kernel(srcg2d, dsth2d, y4)
    return _epilogue(agg4, hdst, b1, W2, b2)


# speedup vs baseline: 1.0901x; 1.0901x over previous
import functools

import jax
import jax.numpy as jnp
from jax import lax
from jax.experimental import pallas as pl
from jax.experimental.pallas import tpu as pltpu
from jax.experimental.pallas import tpu_sc as plsc

N = 10000
E = 160000
D_IN = 256
D_H = 256
DQ = 64
N_CLASSES = 16

NC = 2
NS = 16
B = 128
RPT = 80
NCHUNK = NS * RPT
EPAD = NCHUNK * B - E
RLAST = RPT - EPAD // B
NDUMP = 8
NSP = N + NDUMP
NBUF = 2
NSTEP = RPT // NBUF
RSTRIPE = 624
NREM = N - RSTRIPE * NS

_mesh = plsc.VectorSubcoreMesh(
    core_axis_name="c", subcore_axis_name="s", num_cores=NC, num_subcores=NS
)


def _zero_stripe(zbuf, sh_ref, s):
    base = s * RSTRIPE
    for k in range(4):
        pltpu.sync_copy(zbuf, sh_ref.at[pl.ds(base + k * B, B)])
    pltpu.sync_copy(zbuf.at[pl.ds(0, RSTRIPE - 4 * B)],
                    sh_ref.at[pl.ds(base + 4 * B, RSTRIPE - 4 * B)])

    @pl.when(s == NS - 1)
    def _():
        pltpu.sync_copy(zbuf.at[pl.ds(0, NREM)], sh_ref.at[pl.ds(NS * RSTRIPE, NREM)])


def _copy_stripe(sh_ref, out_ref, s):
    base = s * RSTRIPE
    pltpu.sync_copy(sh_ref.at[pl.ds(base, RSTRIPE)], out_ref.at[pl.ds(base, RSTRIPE)])

    @pl.when(s == NS - 1)
    def _():
        pltpu.sync_copy(sh_ref.at[pl.ds(NS * RSTRIPE, NREM)],
                        out_ref.at[pl.ds(NS * RSTRIPE, NREM)])


def _fill(ref, rows, val):
    v = jnp.full((16,), val, dtype=ref.dtype)

    def body(i, _):
        ref[i, :] = v
        return 0

    lax.fori_loop(0, rows, body, 0)


@functools.partial(
    pl.kernel,
    out_type=(
        jax.ShapeDtypeStruct((N, 16), jnp.float32),
        jax.ShapeDtypeStruct((N, 16), jnp.float32),
    ),
    mesh=_mesh,
    scratch_types=[
        pltpu.VMEM((RPT, B), jnp.int32),
        pltpu.VMEM((B, 16), jnp.float32),
        pltpu.VMEM((B, 16), jnp.float32),
        pltpu.VMEM_SHARED((NSP, 16), jnp.float32),
    ] + [pltpu.SemaphoreType.DMA] * 8,
)
def _deg_kernel(srch2d, dsth2d, hsrc, hdst, idx_all, ones_v, zeros_v, hist_sh,
                *sems):
    c = lax.axis_index("c")
    s = lax.axis_index("s")

    _fill(ones_v, B, 1.0)
    _fill(zeros_v, B, 0.0)
    _zero_stripe(zeros_v, hist_sh, s)

    def scan_edges(edges_ref, out_ref):
        pltpu.sync_copy(edges_ref.at[pl.ds(s * RPT, RPT)], idx_all)
        plsc.subcore_barrier()

        pend = [None] * 8
        for r in range(RPT):
            p = r % 8
            if pend[p] is not None:
                pend[p].wait()
            pend[p] = pltpu.async_copy(
                ones_v, hist_sh.at[idx_all.at[r]], sems[p], add=True)
        for p in range(8):
            pend[p].wait()

        plsc.subcore_barrier()
        _copy_stripe(hist_sh, out_ref, s)

    @pl.when(c == 0)
    def _():
        scan_edges(srch2d, hsrc)

    @pl.when(c == 1)
    def _():
        scan_edges(dsth2d, hdst)


@functools.partial(
    pl.kernel,
    out_type=jax.ShapeDtypeStruct((4 * N, DQ), jnp.float32),
    mesh=_mesh,
    compiler_params=pltpu.CompilerParams(use_tc_tiling_on_sc=False),
    scratch_types=[
        pltpu.VMEM((RPT, B), jnp.int32),
        pltpu.VMEM((RPT, B), jnp.int32),
    ] + [pltpu.VMEM((B, DQ), jnp.float32)] * 7
      + [pltpu.VMEM_SHARED((NSP, DQ), jnp.float32)]
      + [pltpu.SemaphoreType.DMA] * 12,
)
def _agg_kernel(srcg2d, dsth2d, y4, agg4, sidx_all, didx_all, *scr):
    rows = list(scr[0:6])
    zrow_v = scr[6]
    agg_sh = scr[7]
    gsem = list(scr[8:14])
    ssem = list(scr[14:20])
    c = lax.axis_index("c")
    s = lax.axis_index("s")

    def zc(i, _):
        for k in range(DQ // 16):
            zrow_v[i, pl.ds(k * 16, 16)] = jnp.zeros((16,), jnp.float32)
        return 0

    lax.fori_loop(0, B, zc, 0)

    pltpu.sync_copy(dsth2d.at[pl.ds(s * RPT, RPT)], didx_all)

    def gather(r, q):
        return pltpu.async_copy(y4.at[sidx_all.at[r]], rows[q], gsem[q])

    def scat(r, q):
        return pltpu.async_copy(
            rows[q], agg_sh.at[didx_all.at[r]], ssem[q], add=True)

    for p in range(2):
        _zero_stripe(zrow_v, agg_sh, s)

        pltpu.sync_copy(srcg2d.at[pl.ds(s * RPT, RPT)], sidx_all)
        off = (2 * c + p) * N

        def addoff(i, _):
            for k in range(B // 16):
                sl = pl.ds(k * 16, 16)
                sidx_all[i, sl] = sidx_all[i, sl] + off
            return 0

        lax.fori_loop(0, RPT, addoff, 0)

        plsc.subcore_barrier()

        gd = [None] * RPT
        sd = [None] * RPT
        for r in range(RPT):
            if r >= 6:
                sd[r - 6].wait()
            gd[r] = gather(r, r % 6)
            if r >= 3:
                gd[r - 3].wait()
                sd[r - 3] = scat(r - 3, (r - 3) % 6)
        for r in range(RPT - 3, RPT):
            gd[r].wait()
            sd[r] = scat(r, r % 6)
        for r in range(RPT - 6, RPT):
            sd[r].wait()

        plsc.subcore_barrier()
        base = s * RSTRIPE
        pltpu.sync_copy(agg_sh.at[pl.ds(base, RSTRIPE)],
                        agg4.at[pl.ds(off + base, RSTRIPE)])

        @pl.when(s == NS - 1)
        def _(off=off):
            pltpu.sync_copy(agg_sh.at[pl.ds(NS * RSTRIPE, NREM)],
                            agg4.at[pl.ds(off + NS * RSTRIPE, NREM)])


_RB = 2000


def _mm_body(xf_ref, w1_ref, hs_ref, y_ref):
    x = jax.lax.dot_general(
        xf_ref[0], w1_ref[0], (((1,), (0,)), ((), ())),
        preferred_element_type=jnp.float32,
        precision=jax.lax.Precision.DEFAULT,
    )
    deg = jnp.maximum(hs_ref[:, 0], 1.0)
    ns = jax.lax.rsqrt(deg)[:, None]
    y_ref[0] = x * ns


def _mm_scale(in_feat, W1, hsrc):
    w1q = W1.reshape(D_IN, 4, DQ).transpose(1, 0, 2)
    y4 = pl.pallas_call(
        _mm_body,
        grid=(4, N // _RB),
        in_specs=[
            pl.BlockSpec((1, _RB, D_IN), lambda j, i: (0, i, 0)),
            pl.BlockSpec((1, D_IN, DQ), lambda j, i: (j, 0, 0)),
            pl.BlockSpec((_RB, 16), lambda j, i: (i, 0)),
        ],
        out_specs=pl.BlockSpec((1, _RB, DQ), lambda j, i: (j, i, 0)),
        out_shape=jax.ShapeDtypeStruct((4, N, DQ), jnp.float32),
    )(in_feat.reshape(1, N, D_IN), w1q, hsrc)
    return y4.reshape(4 * N, DQ)


def _epi_body(a0_ref, a1_ref, a2_ref, a3_ref, hd_ref, b1_ref, w2_ref, b2_ref,
              out_ref, acc_ref):
    i = pl.program_id(0)
    deg = jnp.maximum(hd_ref[:, 0], 1.0)
    nd = jax.lax.rsqrt(deg)[:, None]
    aq = [a0_ref, a1_ref, a2_ref, a3_ref]
    hq = [aq[q][...] * nd + b1_ref[0:1, DQ * q:DQ * (q + 1)] for q in range(4)]
    ss = sum(jnp.sum(h * h, axis=1, keepdims=True) for h in hq)
    inv = 1.0 / jnp.maximum(jnp.sqrt(ss), 1e-12)
    sq = [jax.nn.sigmoid(h * inv) for h in hq]

    @pl.when(i == 0)
    def _():
        acc_ref[...] = jnp.zeros_like(acc_ref)

    for q in range(4):
        acc_ref[0:1, DQ * q:DQ * (q + 1)] += jnp.sum(sq[q], axis=0, keepdims=True)

    @pl.when(i == (N // _RB) - 1)
    def _():
        hg = acc_ref[...] * (1.0 / N)
        out_ref[...] = (
            jax.lax.dot_general(
                hg, w2_ref[...], (((1,), (0,)), ((), ())),
                preferred_element_type=jnp.float32,
                precision=jax.lax.Precision.DEFAULT,
            )
            + b2_ref[...]
        )


def _epilogue(agg4, hdst, b1, W2, b2):
    qspec = pl.BlockSpec((_RB, DQ), lambda i: (i, 0))
    qspecs = [
        pl.BlockSpec((_RB, DQ), lambda i, q=q: (q * (N // _RB) + i, 0))
        for q in range(4)
    ]
    return pl.pallas_call(
        _epi_body,
        grid=(N // _RB,),
        in_specs=qspecs + [
            pl.BlockSpec((_RB, 16), lambda i: (i, 0)),
            pl.BlockSpec((1, D_H), lambda i: (0, 0)),
            pl.BlockSpec((D_H, N_CLASSES), lambda i: (0, 0)),
            pl.BlockSpec((1, N_CLASSES), lambda i: (0, 0)),
        ],
        out_specs=pl.BlockSpec((1, N_CLASSES), lambda i: (0, 0)),
        out_shape=jax.ShapeDtypeStruct((1, N_CLASSES), jnp.float32),
        scratch_shapes=[pltpu.VMEM((1, D_H), jnp.float32)],
    )(agg4, agg4, agg4, agg4, hdst, b1.reshape(1, D_H), W2,
      b2.reshape(1, N_CLASSES))


def kernel(in_feat, edge_index, W1, b1, W2, b2):
    ar = jnp.arange(EPAD, dtype=jnp.int32)
    dump = N + (ar % NDUMP)
    srch2d = jnp.concatenate([edge_index[0], dump]).reshape(NCHUNK, B)
    srcg2d = jnp.concatenate([edge_index[0], ar % B]).reshape(NCHUNK, B)
    dsth2d = jnp.concatenate([edge_index[1], dump]).reshape(NCHUNK, B)
    hsrc, hdst = _deg_kernel(srch2d, dsth2d)
    y4 = _mm_scale(in_feat, W1, hsrc)
    agg4 = _agg_kernel(srcg2d, dsth2d, y4)
    return _epilogue(agg4, hdst, b1, W2, b2)
